# baseline (speedup 1.0000x reference)
import functools

import jax
import jax.numpy as jnp
from jax import lax
from jax.experimental import pallas as pl
from jax.experimental.pallas import tpu as pltpu
from jax.experimental.pallas import tpu_sc as plsc

_V = 1000
_VP = 1024
_QW = 256
_H = 128
_NB = 4096
_L = 20
_NC = 2
_NS = 16
_NW = _NC * _NS
_BC = _NB // _NW


def _mm_body(emb_ref, w_ref, b_ref, o0, o1, o2, o3):
    for i, o in enumerate((o0, o1, o2, o3)):
        o[...] = lax.dot_general(
            emb_ref[...], w_ref[pl.ds(_QW * i, _QW), :],
            (((1,), (1,)), ((), ())),
            preferred_element_type=jnp.float32,
        ) + b_ref[0, pl.ds(_QW * i, _QW)][None, :]


def _make_tables(emb_table, lin_w_pad, lin_b_pad):
    return pl.pallas_call(
        _mm_body,
        out_shape=[jax.ShapeDtypeStruct((_V, _QW), jnp.float32)] * 4,
    )(emb_table, lin_w_pad, lin_b_pad)


@functools.lru_cache(maxsize=1)
def _make_gather():
    mesh = plsc.VectorSubcoreMesh(core_axis_name="c", subcore_axis_name="s")
    lanes = 16

    @functools.partial(
        pl.kernel,
        mesh=mesh,
        out_type=jax.ShapeDtypeStruct((_L, _V, _NB), jnp.float32),
        scratch_types=[
            pltpu.VMEM((_BC,), jnp.int32),
            pltpu.VMEM((_BC, _QW), jnp.float32),
            pltpu.VMEM((1, _QW, _BC), jnp.float32),
            pltpu.SemaphoreType.DMA,
        ],
        compiler_params=pltpu.CompilerParams(
            use_tc_tiling_on_sc=True, needs_layout_passes=False),
    )
    def _gather_t(m0, m1, m2, m3, idx_hbm, out_hbm, idx_v, rows_v, tbuf, sem):
        wid = lax.axis_index("s") * _NC + lax.axis_index("c")
        b0 = wid * _BC
        iota = lax.iota(jnp.int32, lanes)

        def per_l(l, carry):
            pltpu.sync_copy(idx_hbm.at[pl.ds(l * _NB + b0, _BC)], idx_v)
            for q, m_q in enumerate((m0, m1, m2, m3)):
                pltpu.async_copy(m_q.at[idx_v], rows_v, sem).wait()

                def per_v(v, c2):
                    for j in range(_BC // lanes):
                        vec = plsc.load_gather(
                            rows_v,
                            [iota + (j * lanes), jnp.full((lanes,), v, jnp.int32)],
                        )
                        tbuf[0, v, pl.ds(j * lanes, lanes)] = vec
                    return c2

                lax.fori_loop(0, _QW, per_v, 0)
                n_v = _QW if q < 3 else _V - 3 * _QW
                pltpu.sync_copy(
                    tbuf.at[pl.ds(0, 1), pl.ds(0, n_v)],
                    out_hbm.at[pl.ds(l, 1), pl.ds(q * _QW, n_v), pl.ds(b0, _BC)],
                )
            return carry

        lax.fori_loop(0, _L, per_l, 0)

    return _gather_t


def kernel(input_ids, emb_table, lin_w, lin_b):
    w_pad = jnp.pad(lin_w, ((0, _VP - _V), (0, 0)))
    b_pad = jnp.pad(lin_b, (0, _VP - _V)).reshape(1, _VP)
    tables = _make_tables(emb_table, w_pad, b_pad)
    ids_t = input_ids.T.reshape(-1).astype(jnp.int32)
    out = _make_gather()(*tables, ids_t)
    return out.transpose(2, 0, 1)


# speedup vs baseline: 1.6872x; 1.6872x over previous
import functools

import jax
import jax.numpy as jnp
from jax import lax
from jax.experimental import pallas as pl
from jax.experimental.pallas import tpu as pltpu
from jax.experimental.pallas import tpu_sc as plsc

_V = 1000
_VP = 1024
_QW = 256
_H = 128
_NB = 4096
_L = 20
_NC = 2
_NS = 16
_NW = _NC * _NS
_BC = _NB // _NW


def _mm_body(emb_ref, w_ref, b_ref, o0, o1, o2, o3):
    for i, o in enumerate((o0, o1, o2, o3)):
        o[...] = lax.dot_general(
            emb_ref[...], w_ref[pl.ds(_QW * i, _QW), :],
            (((1,), (1,)), ((), ())),
            preferred_element_type=jnp.float32,
        ) + b_ref[0, pl.ds(_QW * i, _QW)][None, :]


def _make_tables(emb_table, lin_w_pad, lin_b_pad):
    return pl.pallas_call(
        _mm_body,
        out_shape=[jax.ShapeDtypeStruct((_V, _QW), jnp.float32)] * 4,
    )(emb_table, lin_w_pad, lin_b_pad)


@functools.lru_cache(maxsize=1)
def _make_gather():
    mesh = plsc.VectorSubcoreMesh(core_axis_name="c", subcore_axis_name="s")
    lanes = 16

    @functools.partial(
        pl.kernel,
        mesh=mesh,
        out_type=jax.ShapeDtypeStruct((_L, _V, _NB), jnp.float32),
        scratch_types=[
            pltpu.VMEM((_BC,), jnp.int32),
            pltpu.VMEM((_BC, _QW), jnp.float32),
            pltpu.VMEM((1, _QW, _BC), jnp.float32),
            pltpu.SemaphoreType.DMA,
        ],
        compiler_params=pltpu.CompilerParams(
            use_tc_tiling_on_sc=True, needs_layout_passes=False),
    )
    def _gather_t(m0, m1, m2, m3, idx_hbm, out_hbm, idx_v, rows_v, tbuf, sem):
        wid = lax.axis_index("s") * _NC + lax.axis_index("c")
        b0 = wid * _BC
        iota = lax.iota(jnp.int32, lanes)
        b_idx = [iota + (j * lanes) for j in range(_BC // lanes)]

        def per_l(l, carry):
            pltpu.sync_copy(idx_hbm.at[pl.ds(l * _NB + b0, _BC)], idx_v)
            for q, m_q in enumerate((m0, m1, m2, m3)):
                pltpu.async_copy(m_q.at[idx_v], rows_v, sem).wait()

                @plsc.parallel_loop(0, _QW, 1, unroll=4)
                def per_v(v):
                    vs = jnp.full((lanes,), v, jnp.int32)
                    for j in range(_BC // lanes):
                        vec = plsc.load_gather(rows_v, [b_idx[j], vs])
                        tbuf[0, v, pl.ds(j * lanes, lanes)] = vec

                n_v = _QW if q < 3 else _V - 3 * _QW
                pltpu.sync_copy(
                    tbuf.at[pl.ds(0, 1), pl.ds(0, n_v)],
                    out_hbm.at[pl.ds(l, 1), pl.ds(q * _QW, n_v), pl.ds(b0, _BC)],
                )
            return carry

        lax.fori_loop(0, _L, per_l, 0)

    return _gather_t


def kernel(input_ids, emb_table, lin_w, lin_b):
    w_pad = jnp.pad(lin_w, ((0, _VP - _V), (0, 0)))
    b_pad = jnp.pad(lin_b, (0, _VP - _V)).reshape(1, _VP)
    tables = _make_tables(emb_table, w_pad, b_pad)
    ids_t = input_ids.T.reshape(-1).astype(jnp.int32)
    out = _make_gather()(*tables, ids_t)
    return out.transpose(2, 0, 1)


# speedup vs baseline: 2.0657x; 1.2243x over previous
import functools

import jax
import jax.numpy as jnp
from jax import lax
from jax.experimental import pallas as pl
from jax.experimental.pallas import tpu as pltpu
from jax.experimental.pallas import tpu_sc as plsc

_V = 1000
_VP = 1024
_NQ = 8
_QW = 128
_H = 128
_NB = 4096
_L = 20
_NC = 2
_NS = 16
_NW = _NC * _NS
_BC = _NB // _NW
_LANES = 16


def _mm_body(*refs):
    emb_ref, w_ref, b_ref = refs[:3]
    outs = refs[3:]
    for i, o in enumerate(outs):
        o[...] = lax.dot_general(
            emb_ref[...], w_ref[pl.ds(_QW * i, _QW), :],
            (((1,), (1,)), ((), ())),
            preferred_element_type=jnp.float32,
        ) + b_ref[0, pl.ds(_QW * i, _QW)][None, :]


def _make_tables(emb_table, lin_w_pad, lin_b_pad):
    return pl.pallas_call(
        _mm_body,
        out_shape=[jax.ShapeDtypeStruct((_V, _QW), jnp.float32)] * _NQ,
    )(emb_table, lin_w_pad, lin_b_pad)


def _q_rows(q):
    return _QW if q < _NQ - 1 else _V - (_NQ - 1) * _QW


@functools.lru_cache(maxsize=1)
def _make_gather():
    mesh = plsc.VectorSubcoreMesh(core_axis_name="c", subcore_axis_name="s")

    @functools.partial(
        pl.kernel,
        mesh=mesh,
        out_type=jax.ShapeDtypeStruct((_L, _V, _NB), jnp.float32),
        scratch_types=[
            pltpu.VMEM((_BC,), jnp.int32),
            pltpu.VMEM((_BC, _QW), jnp.float32),
            pltpu.VMEM((_BC, _QW), jnp.float32),
            pltpu.VMEM((1, _QW, _BC), jnp.float32),
            pltpu.VMEM((1, _QW, _BC), jnp.float32),
            pltpu.SemaphoreType.DMA,
            pltpu.SemaphoreType.DMA,
            pltpu.SemaphoreType.DMA,
            pltpu.SemaphoreType.DMA,
        ],
        compiler_params=pltpu.CompilerParams(
            use_tc_tiling_on_sc=True, needs_layout_passes=False),
    )
    def _gather_t(*refs):
        tables = refs[:_NQ]
        idx_hbm, out_hbm = refs[_NQ], refs[_NQ + 1]
        idx_v = refs[_NQ + 2]
        rows = refs[_NQ + 3:_NQ + 5]
        tbufs = refs[_NQ + 5:_NQ + 7]
        gsems = refs[_NQ + 7:_NQ + 9]
        ssems = refs[_NQ + 9:_NQ + 11]

        wid = lax.axis_index("s") * _NC + lax.axis_index("c")
        b0 = wid * _BC
        iota = lax.iota(jnp.int32, _LANES)
        b_idx = [iota + (j * _LANES) for j in range(_BC // _LANES)]

        def _store_slices(q, p):
            n_v = _q_rows(q)
            src = tbufs[p].at[pl.ds(0, 1), pl.ds(0, n_v)]
            return src, n_v

        def _transpose(p):
            @plsc.parallel_loop(0, _QW, 1, unroll=4)
            def per_v(v):
                vs = jnp.full((_LANES,), v, jnp.int32)
                for j in range(_BC // _LANES):
                    vec = plsc.load_gather(rows[p], [b_idx[j], vs])
                    tbufs[p][0, v, pl.ds(j * _LANES, _LANES)] = vec

        def per_l(l, carry):
            pltpu.sync_copy(idx_hbm.at[pl.ds(l * _NB + b0, _BC)], idx_v)
            g_cur = pltpu.async_copy(tables[0].at[idx_v], rows[0], gsems[0])
            for q in range(_NQ):
                p = q % 2
                if q + 1 < _NQ:
                    g_next = pltpu.async_copy(
                        tables[q + 1].at[idx_v], rows[1 - p], gsems[1 - p])
                g_cur.wait()
                src_prev, n_prev = _store_slices((q - 2) % _NQ, p)
                dst_prev = out_hbm.at[
                    pl.ds(0, 1), pl.ds(0, n_prev), pl.ds(b0, _BC)]
                if q >= 2:
                    pltpu.make_async_copy(src_prev, dst_prev, ssems[p]).wait()
                else:
                    @pl.when(l > 0)
                    def _():
                        pltpu.make_async_copy(
                            src_prev, dst_prev, ssems[p]).wait()
                _transpose(p)
                src, n_v = _store_slices(q, p)
                pltpu.async_copy(
                    src,
                    out_hbm.at[
                        pl.ds(l, 1), pl.ds(q * _QW, n_v), pl.ds(b0, _BC)],
                    ssems[p],
                )
                if q + 1 < _NQ:
                    g_cur = g_next
            return carry

        lax.fori_loop(0, _L, per_l, 0)
        for q in (_NQ - 2, _NQ - 1):
            p = q % 2
            src, n_v = _store_slices(q, p)
            dst = out_hbm.at[pl.ds(0, 1), pl.ds(0, n_v), pl.ds(b0, _BC)]
            pltpu.make_async_copy(src, dst, ssems[p]).wait()

    return _gather_t


def kernel(input_ids, emb_table, lin_w, lin_b):
    w_pad = jnp.pad(lin_w, ((0, _VP - _V), (0, 0)))
    b_pad = jnp.pad(lin_b, (0, _VP - _V)).reshape(1, _VP)
    tables = _make_tables(emb_table, w_pad, b_pad)
    ids_t = input_ids.T.reshape(-1).astype(jnp.int32)
    out = _make_gather()(*tables, ids_t)
    return out.transpose(2, 0, 1)


# speedup vs baseline: 8.1628x; 3.9515x over previous
import functools

import jax
import jax.numpy as jnp
from jax import lax
from jax.experimental import pallas as pl
from jax.experimental.pallas import tpu as pltpu
from jax.experimental.pallas import tpu_sc as plsc

_V = 1000
_VP = 1024
_H = 128
_NB = 4096
_L = 20
_NC = 2
_NS = 16
_NW = _NC * _NS
_TPW = 4
_NT = _V // 8
_LANES = 16
_G = _NB // _LANES


def _mm_body(w_ref, emb_ref, b_ref, out_ref):
    out_ref[...] = lax.dot_general(
        w_ref[...], emb_ref[...], (((1,), (1,)), ((), ())),
        preferred_element_type=jnp.float32) + b_ref[...]


def _make_table_t(lin_w, emb_pad, lin_b_col):
    return pl.pallas_call(
        _mm_body,
        out_shape=jax.ShapeDtypeStruct((_V, _VP), jnp.float32),
    )(lin_w, emb_pad, lin_b_col)


@functools.lru_cache(maxsize=1)
def _make_lookup():
    mesh = plsc.VectorSubcoreMesh(core_axis_name="c", subcore_axis_name="s")

    @functools.partial(
        pl.kernel,
        mesh=mesh,
        out_type=jax.ShapeDtypeStruct((_L, _V, _NB), jnp.float32),
        scratch_types=[
            pltpu.VMEM((8, _VP), jnp.float32),
            pltpu.VMEM((_NB,), jnp.int32),
            pltpu.VMEM((1, 8, _NB), jnp.float32),
            pltpu.VMEM((1, 8, _NB), jnp.float32),
            pltpu.SemaphoreType.DMA,
            pltpu.SemaphoreType.DMA,
        ],
        compiler_params=pltpu.CompilerParams(
            use_tc_tiling_on_sc=True, needs_layout_passes=False),
    )
    def _lookup(mt_hbm, idx_hbm, out_hbm, slab, ids_v, ob0, ob1, sem0, sem1):
        wid = lax.axis_index("s") * _NC + lax.axis_index("c")
        obufs = (ob0, ob1)
        sems = (sem0, sem1)

        def _wait_store(p):
            pltpu.make_async_copy(
                obufs[p],
                out_hbm.at[pl.ds(0, 1), pl.ds(0, 8), :],
                sems[p],
            ).wait()

        def _do_unit(l, t, p, first_k):
            if first_k:
                @pl.when(l > 1)
                def _():
                    _wait_store(p)
            else:
                _wait_store(p)
            pltpu.sync_copy(idx_hbm.at[pl.ds(l * _NB, _NB)], ids_v)
            ob = obufs[p]

            @plsc.parallel_loop(0, _G, 1, unroll=4)
            def per_g(g):
                idv = ids_v[pl.ds(g * _LANES, _LANES)]
                for v in range(8):
                    vec = plsc.load_gather(
                        slab, [jnp.full((_LANES,), v, jnp.int32), idv])
                    ob[0, v, pl.ds(g * _LANES, _LANES)] = vec

            pltpu.async_copy(
                ob,
                out_hbm.at[pl.ds(l, 1), pl.ds(8 * t, 8), :],
                sems[p],
            )

        for k in range(_TPW):
            t = jnp.minimum(_TPW * wid + k, _NT - 1)
            pltpu.sync_copy(mt_hbm.at[pl.ds(8 * t, 8), :], slab)

            def per_l2(l2, carry):
                l = l2 * 2
                _do_unit(l, t, 0, k == 0)
                _do_unit(l + 1, t, 1, k == 0)
                return carry

            lax.fori_loop(0, _L // 2, per_l2, 0)

        _wait_store(0)
        _wait_store(1)

    return _lookup


def kernel(input_ids, emb_table, lin_w, lin_b):
    emb_pad = jnp.pad(emb_table, ((0, _VP - _V), (0, 0)))
    b_col = lin_b.reshape(_V, 1)
    mt = _make_table_t(lin_w, emb_pad, b_col)
    ids_t = input_ids.T.reshape(-1).astype(jnp.int32)
    out = _make_lookup()(mt, ids_t)
    return out.transpose(2, 0, 1)


# speedup vs baseline: 10.5751x; 1.2955x over previous
import functools

import jax
import jax.numpy as jnp
from jax import lax
from jax.experimental import pallas as pl
from jax.experimental.pallas import tpu as pltpu
from jax.experimental.pallas import tpu_sc as plsc

_V = 1000
_VP = 1024
_H = 128
_NB = 4096
_L = 20
_NC = 2
_NS = 16
_NW = _NC * _NS
_TPW = 4
_NT = _V // 8
_LANES = 16
_G = _NB // _LANES


def _mm_body(w_ref, emb_ref, b_ref, out_ref):
    out_ref[...] = lax.dot_general(
        w_ref[...], emb_ref[...], (((1,), (1,)), ((), ())),
        preferred_element_type=jnp.float32) + b_ref[...]


def _make_table_t(lin_w, emb_pad, lin_b_col):
    return pl.pallas_call(
        _mm_body,
        out_shape=jax.ShapeDtypeStruct((_V, _VP), jnp.float32),
    )(lin_w, emb_pad, lin_b_col)


@functools.lru_cache(maxsize=1)
def _make_lookup():
    mesh = plsc.VectorSubcoreMesh(core_axis_name="c", subcore_axis_name="s")

    @functools.partial(
        pl.kernel,
        mesh=mesh,
        out_type=jax.ShapeDtypeStruct((_L, _V, _NB), jnp.float32),
        scratch_types=[
            pltpu.VMEM((8 * _TPW, _VP), jnp.float32),
            pltpu.VMEM((_NB,), jnp.int32),
            pltpu.VMEM((1, 8, _NB), jnp.float32),
            pltpu.VMEM((1, 8, _NB), jnp.float32),
            pltpu.SemaphoreType.DMA,
            pltpu.SemaphoreType.DMA,
        ],
        compiler_params=pltpu.CompilerParams(
            use_tc_tiling_on_sc=True, needs_layout_passes=False),
    )
    def _lookup(mt_hbm, idx_hbm, out_hbm, slabs, ids_v, ob0, ob1, sem0, sem1):
        wid = lax.axis_index("s") * _NC + lax.axis_index("c")
        obufs = (ob0, ob1)
        sems = (sem0, sem1)

        def _wait_store(p):
            pltpu.make_async_copy(
                obufs[p],
                out_hbm.at[pl.ds(0, 1), pl.ds(0, 8), :],
                sems[p],
            ).wait()

        ts = []
        for k in range(_TPW):
            t = jnp.minimum(_TPW * wid + k, _NT - 1)
            ts.append(t)
            pltpu.sync_copy(mt_hbm.at[pl.ds(8 * t, 8), :],
                            slabs.at[pl.ds(8 * k, 8), :])

        def per_l(l, carry):
            pltpu.sync_copy(idx_hbm.at[pl.ds(l * _NB, _NB)], ids_v)
            for k in range(_TPW):
                p = k % 2
                if k < 2:
                    @pl.when(l > 0)
                    def _():
                        _wait_store(p)
                else:
                    _wait_store(p)
                ob = obufs[p]

                @plsc.parallel_loop(0, _G, 1, unroll=4)
                def per_g(g):
                    idv = ids_v[pl.ds(g * _LANES, _LANES)]
                    for v in range(8):
                        vec = plsc.load_gather(
                            slabs,
                            [jnp.full((_LANES,), 8 * k + v, jnp.int32), idv])
                        ob[0, v, pl.ds(g * _LANES, _LANES)] = vec

                pltpu.async_copy(
                    ob,
                    out_hbm.at[pl.ds(l, 1), pl.ds(8 * ts[k], 8), :],
                    sems[p],
                )
            return carry

        lax.fori_loop(0, _L, per_l, 0)
        _wait_store(0)
        _wait_store(1)

    return _lookup


def kernel(input_ids, emb_table, lin_w, lin_b):
    emb_pad = jnp.pad(emb_table, ((0, _VP - _V), (0, 0)))
    b_col = lin_b.reshape(_V, 1)
    mt = _make_table_t(lin_w, emb_pad, b_col)
    ids_t = input_ids.T.reshape(-1).astype(jnp.int32)
    out = _make_lookup()(mt, ids_t)
    return out.transpose(2, 0, 1)


# speedup vs baseline: 12.0977x; 1.1440x over previous
import functools

import jax
import jax.numpy as jnp
from jax import lax
from jax.experimental import pallas as pl
from jax.experimental.pallas import tpu as pltpu
from jax.experimental.pallas import tpu_sc as plsc

_V = 1000
_VP = 1024
_H = 128
_NB = 4096
_L = 20
_NC = 2
_NS = 16
_NW = _NC * _NS
_TPW = 4
_NT = _V // 8
_LANES = 16
_G = _NB // _LANES


def _mm_body(w_ref, emb_ref, b_ref, out_ref):
    out_ref[...] = lax.dot_general(
        w_ref[...], emb_ref[...], (((1,), (1,)), ((), ())),
        preferred_element_type=jnp.float32) + b_ref[...]


def _make_table_t(lin_w, emb_pad, lin_b_col):
    return pl.pallas_call(
        _mm_body,
        out_shape=jax.ShapeDtypeStruct((_V, _VP), jnp.float32),
    )(lin_w, emb_pad, lin_b_col)


@functools.lru_cache(maxsize=1)
def _make_lookup():
    mesh = plsc.VectorSubcoreMesh(core_axis_name="c", subcore_axis_name="s")

    @functools.partial(
        pl.kernel,
        mesh=mesh,
        out_type=jax.ShapeDtypeStruct((_L, _V, _NB), jnp.float32),
        scratch_types=[
            pltpu.VMEM((8 * _TPW, _VP), jnp.float32),
            pltpu.VMEM((_NB,), jnp.int32),
            pltpu.VMEM((_NB,), jnp.int32),
            pltpu.VMEM((1, 8, _NB), jnp.float32),
            pltpu.VMEM((1, 8, _NB), jnp.float32),
            pltpu.SemaphoreType.DMA,
            pltpu.SemaphoreType.DMA,
            pltpu.SemaphoreType.DMA,
        ],
        compiler_params=pltpu.CompilerParams(
            use_tc_tiling_on_sc=True, needs_layout_passes=False),
    )
    def _lookup(mt_hbm, idx_hbm, out_hbm, slabs, ib0, ib1, ob0, ob1,
                sem0, sem1, isem):
        wid = lax.axis_index("s") * _NC + lax.axis_index("c")
        obufs = (ob0, ob1)
        sems = (sem0, sem1)
        ibufs = (ib0, ib1)

        def _wait_store(p):
            pltpu.make_async_copy(
                obufs[p],
                out_hbm.at[pl.ds(0, 1), pl.ds(0, 8), :],
                sems[p],
            ).wait()

        ts = []
        for k in range(_TPW):
            t = jnp.minimum(_TPW * wid + k, _NT - 1)
            ts.append(t)
            pltpu.sync_copy(mt_hbm.at[pl.ds(8 * t, 8), :],
                            slabs.at[pl.ds(8 * k, 8), :])

        def _units(l, ids_v):
            for k in range(_TPW):
                p = k % 2
                if k < 2:
                    @pl.when(l > 0)
                    def _():
                        _wait_store(p)
                else:
                    _wait_store(p)
                ob = obufs[p]

                @plsc.parallel_loop(0, _G, 1, unroll=8)
                def per_g(g):
                    idv = ids_v[pl.ds(g * _LANES, _LANES)]
                    for v in range(8):
                        vec = plsc.load_gather(
                            slabs,
                            [jnp.full((_LANES,), 8 * k + v, jnp.int32), idv])
                        ob[0, v, pl.ds(g * _LANES, _LANES)] = vec

                pltpu.async_copy(
                    ob,
                    out_hbm.at[pl.ds(l, 1), pl.ds(8 * ts[k], 8), :],
                    sems[p],
                )

        pltpu.sync_copy(idx_hbm.at[pl.ds(0, _NB)], ib0)

        def per_l2(l2, carry):
            l = l2 * 2
            a1 = pltpu.async_copy(
                idx_hbm.at[pl.ds((l + 1) * _NB, _NB)], ib1, isem)
            _units(l, ib0)
            a1.wait()
            nxt = jnp.minimum(l + 2, _L - 1) * _NB
            a0 = pltpu.async_copy(idx_hbm.at[pl.ds(nxt, _NB)], ib0, isem)
            _units(l + 1, ib1)
            a0.wait()
            return carry

        lax.fori_loop(0, _L // 2, per_l2, 0)
        _wait_store(0)
        _wait_store(1)

    return _lookup


def kernel(input_ids, emb_table, lin_w, lin_b):
    emb_pad = jnp.pad(emb_table, ((0, _VP - _V), (0, 0)))
    b_col = lin_b.reshape(_V, 1)
    mt = _make_table_t(lin_w, emb_pad, b_col)
    ids_t = input_ids.T.reshape(-1).astype(jnp.int32)
    out = _make_lookup()(mt, ids_t)
    return out.transpose(2, 0, 1)
